# Initial kernel scaffold; baseline (speedup 1.0000x reference)
import jax
import jax.numpy as jnp
from jax.experimental import pallas as pl


def kernel(encoder_outputs, encoder_hidden, emb_W, emb_b, Wa_W, Wa_b, Ua_W, Ua_b, Va_W, Va_b, Wih, Whh, bih, bhh, out_W, out_b):
    raise NotImplementedError("write your pallas kernel here")


# speedup vs baseline: 4.9840x; 4.9840x over previous
import jax
import jax.numpy as jnp
from jax.experimental import pallas as pl
from jax.experimental.pallas import tpu as pltpu


def _decode_kernel(keysT_ref, h0T_ref, emb_col_ref, emb_b_ref, Wa_ref,
                   Wa_b_ref, Ua_ref, Ua_b_ref, Va_col_ref, Va_b_ref, Wih_ref,
                   Whh_ref, bih_ref, bhh_ref, outW_ref, outb_ref, attn_ref, chosen_ref,
                   logp_ref, uk_ref, keyslp_ref, mask_ref, oh_ref, dec_ref):
    Sn, Hn, Bn = keysT_ref.shape
    f32 = jnp.float32

    Ua = Ua_ref[...]
    Ua_b = Ua_b_ref[...]

    def proj(s, _):
        ks = keysT_ref[pl.ds(s, 1)].reshape(Hn, Bn)
        uk_ref[pl.ds(s, 1)] = (jnp.dot(Ua, ks, preferred_element_type=f32)
                               + Ua_b)[None]
        keyslp_ref[pl.ds(s, 1)] = ks.astype(jnp.bfloat16).astype(f32)[None]
        return 0

    jax.lax.fori_loop(0, Sn, proj, 0)

    emb_col = emb_col_ref[...]
    emb_b = emb_b_ref[...]
    Wa = Wa_ref[...]
    Wa_b = Wa_b_ref[...]
    va3 = Va_col_ref[...].astype(jnp.bfloat16).astype(f32)[None]
    va_b = Va_b_ref[...][None]
    Wih = Wih_ref[...]
    Whh = Whh_ref[...]
    bih = bih_ref[...]
    bhh = bhh_ref[...]
    outW = outW_ref[...]
    outb = outb_ref[...]
    neg_inf = f32(-jnp.inf)
    iota_s = jax.lax.broadcasted_iota(jnp.int32, (Sn, Bn), 0)

    mask_ref[...] = jnp.zeros((Sn, Bn), f32)
    oh_ref[...] = (iota_s == 0).astype(f32)
    dec_ref[...] = jnp.ones((1, Bn), f32)

    def body(i, hT):
        maskT = mask_ref[...]
        chosen_ohT = oh_ref[...]
        decT = dec_ref[...]
        embT = emb_col * decT + emb_b
        qT = jnp.dot(Wa, hT, preferred_element_type=f32) + Wa_b
        e = jnp.tanh(uk_ref[...] + qT[None])
        e_lp = e.astype(jnp.bfloat16).astype(f32)
        scores3 = jnp.sum(e_lp * va3, axis=1, keepdims=True) + va_b
        m3 = jnp.max(scores3, axis=0, keepdims=True)
        ex3 = jnp.exp(scores3 - m3)
        w3 = ex3 / jnp.sum(ex3, axis=0, keepdims=True)
        w_lp = w3.astype(jnp.bfloat16).astype(f32)
        ctxT = jnp.sum(w_lp * keyslp_ref[...], axis=0)
        x2T = jnp.concatenate([embT, ctxT], axis=0)
        gi = jnp.dot(Wih, x2T, preferred_element_type=f32) + bih
        gh = jnp.dot(Whh, hT, preferred_element_type=f32) + bhh
        r = jax.nn.sigmoid(gi[:Hn] + gh[:Hn])
        z = jax.nn.sigmoid(gi[Hn:2 * Hn] + gh[Hn:2 * Hn])
        n = jnp.tanh(gi[2 * Hn:] + r * gh[2 * Hn:])
        h_newT = (1.0 - z) * n + z * hT
        logitsT = jnp.dot(outW, h_newT, preferred_element_type=f32) + outb

        ml = jnp.max(logitsT, axis=0, keepdims=True)
        idx_un = jnp.min(jnp.where(logitsT == ml, iota_s, Sn), axis=0,
                         keepdims=True)
        dec_newT = idx_un.astype(f32)

        new_mask = jnp.where(chosen_ohT > 0.5, neg_inf, maskT)
        first = i == 0
        masked_first = jnp.where(iota_s == 0, f32(1.0), neg_inf)
        masked = jnp.where(first, masked_first, logitsT + new_mask)
        mask_out = jnp.where(first, maskT, new_mask)
        m2 = jnp.max(masked, axis=0, keepdims=True)
        idx2 = jnp.min(jnp.where(masked == m2, iota_s, Sn), axis=0,
                       keepdims=True)
        oh_new = (iota_s == idx2).astype(f32)
        logp_col = 0.0 - jnp.log(jnp.sum(jnp.exp(masked - m2), axis=0,
                                         keepdims=True))

        attn_ref[:, pl.ds(i, 1), :] = w3
        chosen_ref[pl.ds(i, 1)] = idx2.astype(f32)[None]
        logp_ref[pl.ds(i, 1)] = logp_col[None]
        mask_ref[...] = mask_out
        oh_ref[...] = oh_new
        dec_ref[...] = dec_newT
        return h_newT

    jax.lax.fori_loop(0, Sn, body, h0T_ref[...])


@jax.jit
def kernel(encoder_outputs, encoder_hidden, emb_W, emb_b, Wa_W, Wa_b, Ua_W,
           Ua_b, Va_W, Va_b, Wih, Whh, bih, bhh, out_W, out_b):
    Bn, Sn, Hn = encoder_outputs.shape
    keysT = jnp.transpose(encoder_outputs, (1, 2, 0))
    h0T = encoder_hidden[0].T

    attn, chosen, logp = pl.pallas_call(
        _decode_kernel,
        out_shape=[
            jax.ShapeDtypeStruct((Sn, Sn, Bn), jnp.float32),
            jax.ShapeDtypeStruct((Sn, 1, Bn), jnp.float32),
            jax.ShapeDtypeStruct((Sn, 1, Bn), jnp.float32),
        ],
        scratch_shapes=[pltpu.VMEM((Sn, Hn, Bn), jnp.float32),
                        pltpu.VMEM((Sn, Hn, Bn), jnp.float32),
                        pltpu.VMEM((Sn, Bn), jnp.float32),
                        pltpu.VMEM((Sn, Bn), jnp.float32),
                        pltpu.VMEM((1, Bn), jnp.float32)],
    )(keysT, h0T, emb_W, emb_b[:, None], Wa_W, Wa_b[:, None], Ua_W,
      Ua_b[:, None], Va_W.T, Va_b[:, None], Wih, Whh, bih[:, None],
      bhh[:, None], out_W, out_b[:, None])

    tours = chosen[:, 0, :].T[:, None, :]
    tour_logp = logp[:, 0, :].T
    attentions = jnp.transpose(attn, (2, 1, 0))
    return tours, tour_logp, attentions


# speedup vs baseline: 6.6308x; 1.3304x over previous
import jax
import jax.numpy as jnp
from jax.experimental import pallas as pl
from jax.experimental.pallas import tpu as pltpu


def _decode_kernel(keysT_ref, keysB_ref, h0T_ref, emb_col_ref, emb_b_ref,
                   Wa_ref, Wa_b_ref, Ua_ref, Ua_b_ref, Va_row_ref, Va_b_ref,
                   Wih_ref, Whh_ref, bih_ref, bhh_ref, outW_ref, outb_ref,
                   attn_ref, chosen_ref, logp_ref,
                   uk_ref, e_ref, sc_ref, ctx_ref, mask_ref, oh_ref, dec_ref):
    Sn, Hn, Bn = keysT_ref.shape
    f32 = jnp.float32

    Ua = Ua_ref[...]
    Ua_b = Ua_b_ref[...]

    def proj(s, _):
        ks = keysT_ref[pl.ds(s, 1)].reshape(Hn, Bn)
        uk_ref[pl.ds(s, 1)] = (jnp.dot(Ua, ks, preferred_element_type=f32)
                               + Ua_b)[None]
        return 0

    jax.lax.fori_loop(0, Sn, proj, 0)

    emb_col = emb_col_ref[...]
    emb_b = emb_b_ref[...]
    Wa = Wa_ref[...]
    Wa_b = Wa_b_ref[...]
    va_row = Va_row_ref[...]
    va_b = Va_b_ref[...]
    Wih = Wih_ref[...]
    Whh = Whh_ref[...]
    bih = bih_ref[...]
    bhh = bhh_ref[...]
    outW = outW_ref[...]
    outb = outb_ref[...]
    neg_inf = f32(-jnp.inf)
    iota_s = jax.lax.broadcasted_iota(jnp.int32, (Sn, Bn), 0)

    mask_ref[...] = jnp.zeros((Sn, Bn), f32)
    oh_ref[...] = (iota_s == 0).astype(f32)
    dec_ref[...] = jnp.ones((1, Bn), f32)

    def tree_sum(x):
        p = []
        for j in range(8):
            acc = x[j:j + 1, :]
            for r in range(1, Sn // 8):
                acc = acc + x[8 * r + j:8 * r + j + 1, :]
            p.append(acc)
        t = [p[j] + p[j + 4] for j in range(4)]
        t2 = [t[j] + t[j + 2] for j in range(2)]
        return t2[0] + t2[1]

    def body(i, hT):
        maskT = mask_ref[...]
        chosen_ohT = oh_ref[...]
        decT = dec_ref[...]
        embT = emb_col * decT + emb_b
        qT = jnp.dot(Wa, hT, preferred_element_type=f32) + Wa_b
        e_ref[...] = jnp.tanh(uk_ref[...] + qT[None])
        for s in range(Sn):
            es = e_ref[pl.ds(s, 1)].reshape(Hn, Bn)
            sc_ref[pl.ds(s, 1), :] = jnp.dot(
                va_row, es, preferred_element_type=f32) + va_b
        scores = sc_ref[...]
        m = jnp.max(scores, axis=0, keepdims=True)
        ex = jnp.exp(scores - m)
        w = ex / tree_sum(ex)
        attn_ref[pl.ds(i, 1)] = w[None]
        wT = jnp.transpose(w)
        for b in range(Bn):
            wb = jax.lax.slice_in_dim(wT, b, b + 1, axis=0)
            kb = keysB_ref[pl.ds(b, 1)].reshape(Sn, Hn)
            ctx_ref[pl.ds(b, 1), :] = jnp.dot(
                wb, kb, preferred_element_type=f32)
        ctxT = jnp.transpose(ctx_ref[...])
        x2T = jnp.concatenate([embT, ctxT], axis=0)
        gi = jnp.dot(Wih, x2T, preferred_element_type=f32) + bih
        gh = jnp.dot(Whh, hT, preferred_element_type=f32) + bhh
        r = jax.nn.sigmoid(gi[:Hn] + gh[:Hn])
        z = jax.nn.sigmoid(gi[Hn:2 * Hn] + gh[Hn:2 * Hn])
        n = jnp.tanh(gi[2 * Hn:] + r * gh[2 * Hn:])
        h_newT = (1.0 - z) * n + z * hT
        logitsT = jnp.dot(outW, h_newT, preferred_element_type=f32) + outb

        ml = jnp.max(logitsT, axis=0, keepdims=True)
        idx_un = jnp.min(jnp.where(logitsT == ml, iota_s, Sn), axis=0,
                         keepdims=True)
        dec_newT = idx_un.astype(f32)

        new_mask = jnp.where(chosen_ohT > 0.5, neg_inf, maskT)
        first = i == 0
        masked_first = jnp.where(iota_s == 0, f32(1.0), neg_inf)
        masked_else = jnp.where(new_mask < -1.0, neg_inf, logitsT)
        masked = jnp.where(first, masked_first, masked_else)
        mask_out = jnp.where(first, maskT, new_mask)
        m2 = jnp.max(masked, axis=0, keepdims=True)
        idx2 = jnp.min(jnp.where(masked == m2, iota_s, Sn), axis=0,
                       keepdims=True)
        oh_new = (iota_s == idx2).astype(f32)
        lsum = tree_sum(jnp.exp(masked - m2))
        logp_col = 0.0 - jnp.log(lsum)

        chosen_ref[pl.ds(i, 1)] = idx2.astype(f32)[None]
        logp_ref[pl.ds(i, 1)] = logp_col[None]
        mask_ref[...] = mask_out
        oh_ref[...] = oh_new
        dec_ref[...] = dec_newT
        return h_newT

    jax.lax.fori_loop(0, Sn, body, h0T_ref[...])


@jax.jit
def kernel(encoder_outputs, encoder_hidden, emb_W, emb_b, Wa_W, Wa_b, Ua_W,
           Ua_b, Va_W, Va_b, Wih, Whh, bih, bhh, out_W, out_b):
    Bn, Sn, Hn = encoder_outputs.shape
    keysT = jnp.transpose(encoder_outputs, (1, 2, 0))
    h0T = encoder_hidden[0].T

    attn, chosen, logp = pl.pallas_call(
        _decode_kernel,
        out_shape=[
            jax.ShapeDtypeStruct((Sn, Sn, Bn), jnp.float32),
            jax.ShapeDtypeStruct((Sn, 1, Bn), jnp.float32),
            jax.ShapeDtypeStruct((Sn, 1, Bn), jnp.float32),
        ],
        scratch_shapes=[pltpu.VMEM((Sn, Hn, Bn), jnp.float32),
                        pltpu.VMEM((Sn, Hn, Bn), jnp.float32),
                        pltpu.VMEM((Sn, Bn), jnp.float32),
                        pltpu.VMEM((Bn, Hn), jnp.float32),
                        pltpu.VMEM((Sn, Bn), jnp.float32),
                        pltpu.VMEM((Sn, Bn), jnp.float32),
                        pltpu.VMEM((1, Bn), jnp.float32)],
    )(keysT, encoder_outputs, h0T, emb_W, emb_b[:, None], Wa_W, Wa_b[:, None],
      Ua_W, Ua_b[:, None], Va_W, Va_b[None], Wih, Whh, bih[:, None],
      bhh[:, None], out_W, out_b[:, None])

    tours = chosen[:, 0, :].T[:, None, :]
    tour_logp = logp[:, 0, :].T
    attentions = jnp.transpose(attn, (2, 0, 1))
    return tours, tour_logp, attentions
